# Initial kernel scaffold; baseline (speedup 1.0000x reference)
import jax
import jax.numpy as jnp
from jax.experimental import pallas as pl


def kernel(token_ids, token_table, pos_table):
    raise NotImplementedError("write your pallas kernel here")


# speedup vs baseline: 1.3920x; 1.3920x over previous
import functools

import jax
import jax.numpy as jnp
from jax import lax
from jax.experimental import pallas as pl
from jax.experimental.pallas import tpu as pltpu
from jax.experimental.pallas import tpu_sc as plsc

D = 32
MAXLEN = 200
NC = 2
NS = 16
NW = NC * NS
K = 80
NSTR = 10
CH = K * NSTR
SEQS_PER_CHUNK = CH // MAXLEN


@functools.lru_cache(maxsize=None)
def _build(n_rows):
    rows_per_worker = n_rows // NW
    n_chunks = rows_per_worker // CH
    mesh = plsc.VectorSubcoreMesh(core_axis_name="c", subcore_axis_name="s")

    @functools.partial(
        pl.kernel,
        mesh=mesh,
        out_type=jax.ShapeDtypeStruct((n_rows, D), jnp.float32),
        scratch_types=[
            pltpu.VMEM((CH,), jnp.int32),
            pltpu.VMEM((CH, D), jnp.float32),
            pltpu.VMEM((MAXLEN, D), jnp.float32),
            pltpu.SemaphoreType.DMA,
        ],
        compiler_params=pltpu.CompilerParams(use_tc_tiling_on_sc=False),
    )
    def emb(ids_hbm, tok_hbm, pos_hbm, out_hbm, idx_v, rows_v, pos_v, sem):
        wid = lax.axis_index("s") * NC + lax.axis_index("c")
        base = wid * rows_per_worker
        pltpu.sync_copy(pos_hbm, pos_v)

        def chunk_body(g, carry):
            start = base + g * CH
            pltpu.sync_copy(ids_hbm.at[pl.ds(start, CH)], idx_v)
            copies = [
                pltpu.async_copy(tok_hbm.at[idx_v.at[pl.ds(j * K, K)]],
                                 rows_v.at[pl.ds(j * K, K)], sem)
                for j in range(NSTR)
            ]
            for cp in copies:
                cp.wait()

            def add_body(r2, c2):
                for u in range(8):
                    r = r2 * 8 + u
                    p0 = pos_v[r, pl.ds(0, 16)]
                    p1 = pos_v[r, pl.ds(16, 16)]
                    for s in range(SEQS_PER_CHUNK):
                        row = s * MAXLEN + r
                        plsc.addupdate(rows_v.at[row, pl.ds(0, 16)], p0)
                        plsc.addupdate(rows_v.at[row, pl.ds(16, 16)], p1)
                return c2

            lax.fori_loop(0, MAXLEN // 8, add_body, 0)
            pltpu.sync_copy(rows_v, out_hbm.at[pl.ds(start, CH)])
            return carry

        lax.fori_loop(0, n_chunks, chunk_body, 0)

    return emb


def kernel(token_ids, token_table, pos_table):
    batch, seq = token_ids.shape
    n_rows = batch * seq
    ids_flat = token_ids.astype(jnp.int32).reshape(n_rows)
    out = _build(n_rows)(ids_flat, token_table, pos_table)
    return out.reshape(batch, seq, D)


# speedup vs baseline: 1.4917x; 1.0716x over previous
import functools

import jax
import jax.numpy as jnp
from jax import lax
from jax.experimental import pallas as pl
from jax.experimental.pallas import tpu as pltpu
from jax.experimental.pallas import tpu_sc as plsc

D = 32
MAXLEN = 200
NC = 2
NS = 16
NW = NC * NS
K = 80
NSTR = 10
CH = K * NSTR
SEQS_PER_CHUNK = CH // MAXLEN


@functools.lru_cache(maxsize=None)
def _build(n_rows):
    rows_per_worker = n_rows // NW
    n_chunks = rows_per_worker // CH
    mesh = plsc.VectorSubcoreMesh(core_axis_name="c", subcore_axis_name="s")

    @functools.partial(
        pl.kernel,
        mesh=mesh,
        out_type=jax.ShapeDtypeStruct((n_rows, D), jnp.float32),
        scratch_types=[
            pltpu.VMEM((CH,), jnp.int32),
            pltpu.VMEM((CH,), jnp.int32),
            pltpu.VMEM((CH, D), jnp.float32),
            pltpu.VMEM((CH, D), jnp.float32),
            pltpu.VMEM((MAXLEN, D), jnp.float32),
            pltpu.SemaphoreType.DMA,
            pltpu.SemaphoreType.DMA,
            pltpu.SemaphoreType.DMA,
            pltpu.SemaphoreType.DMA,
        ],
        compiler_params=pltpu.CompilerParams(use_tc_tiling_on_sc=False),
    )
    def emb(ids_hbm, tok_hbm, pos_hbm, out_hbm,
            idx0, idx1, rows0, rows1, pos_v,
            gsem0, gsem1, isem0, isem1):
        idx = (idx0, idx1)
        rows = (rows0, rows1)
        gsem = (gsem0, gsem1)
        isem = (isem0, isem1)

        wid = lax.axis_index("s") * NC + lax.axis_index("c")
        base = wid * rows_per_worker
        pltpu.sync_copy(pos_hbm, pos_v)

        def fire_gathers(b, start):
            for j in range(NSTR):
                pltpu.async_copy(tok_hbm.at[idx[b].at[pl.ds(j * K, K)]],
                                 rows[b].at[pl.ds(j * K, K)], gsem[b])

        def drain_gathers(b):
            pltpu.make_async_copy(out_hbm.at[pl.ds(0, CH)], rows[b],
                                  gsem[b]).wait()

        def drain_idx(b):
            pltpu.make_async_copy(ids_hbm.at[pl.ds(0, CH)], idx[b],
                                  isem[b]).wait()

        def add_pos(b):
            def add_body(r2, c2):
                for u in range(8):
                    r = r2 * 8 + u
                    p0 = pos_v[r, pl.ds(0, 16)]
                    p1 = pos_v[r, pl.ds(16, 16)]
                    for s in range(SEQS_PER_CHUNK):
                        row = s * MAXLEN + r
                        plsc.addupdate(rows[b].at[row, pl.ds(0, 16)], p0)
                        plsc.addupdate(rows[b].at[row, pl.ds(16, 16)], p1)
                return c2
            lax.fori_loop(0, MAXLEN // 8, add_body, 0)

        pltpu.sync_copy(ids_hbm.at[pl.ds(base, CH)], idx0)
        fire_gathers(0, base)
        pltpu.async_copy(ids_hbm.at[pl.ds(base + CH, CH)], idx1, isem1)

        def pair_body(go, carry):
            for par in range(2):
                b, nb = par, 1 - par
                g = 2 * go + par
                start = base + g * CH
                drain_gathers(b)

                @pl.when(g + 1 < n_chunks)
                def _fire_next():
                    drain_idx(nb)
                    fire_gathers(nb, start + CH)

                @pl.when(g + 2 < n_chunks)
                def _prefetch_idx():
                    pltpu.async_copy(
                        ids_hbm.at[pl.ds(start + 2 * CH, CH)], idx[b],
                        isem[b])

                add_pos(b)
                pltpu.sync_copy(rows[b], out_hbm.at[pl.ds(start, CH)])
            return carry

        lax.fori_loop(0, n_chunks // 2, pair_body, 0)

    return emb


def kernel(token_ids, token_table, pos_table):
    batch, seq = token_ids.shape
    n_rows = batch * seq
    ids_flat = token_ids.astype(jnp.int32).reshape(n_rows)
    out = _build(n_rows)(ids_flat, token_table, pos_table)
    return out.reshape(batch, seq, D)
